# initial kernel scaffold (unmeasured)
import jax
import jax.numpy as jnp
from jax import lax
from jax.experimental import pallas as pl
from jax.experimental.pallas import tpu as pltpu


def kernel(
    x,
):
    def body(*refs):
        pass

    out_shape = jax.ShapeDtypeStruct(..., jnp.float32)
    return pl.pallas_call(body, out_shape=out_shape)(...)


# baseline (device time: 12628 ns/iter reference)
import jax
import jax.numpy as jnp
from jax import lax
from jax.experimental import pallas as pl
from jax.experimental.pallas import tpu as pltpu

N_DEV = 8


def kernel(x):
    m, n = x.shape

    def body(x_ref, out_ref, halo_ref, send_sems, recv_sems):
        my = lax.axis_index("i")
        is_first = my == 0
        is_last = my == N_DEV - 1

        barrier = pltpu.get_barrier_semaphore()

        @pl.when(jnp.logical_not(is_first))
        def _():
            pl.semaphore_signal(
                barrier, inc=1, device_id=(my - 1,),
                device_id_type=pl.DeviceIdType.MESH,
            )

        @pl.when(jnp.logical_not(is_last))
        def _():
            pl.semaphore_signal(
                barrier, inc=1, device_id=(my + 1,),
                device_id_type=pl.DeviceIdType.MESH,
            )

        @pl.when(is_first | is_last)
        def _():
            pl.semaphore_wait(barrier, 1)

        @pl.when(jnp.logical_not(is_first | is_last))
        def _():
            pl.semaphore_wait(barrier, 2)

        send_right = pltpu.make_async_remote_copy(
            src_ref=x_ref.at[pl.ds(m - 1, 1), :],
            dst_ref=halo_ref.at[0],
            send_sem=send_sems.at[0],
            recv_sem=recv_sems.at[0],
            device_id=(my + 1,),
            device_id_type=pl.DeviceIdType.MESH,
        )
        send_left = pltpu.make_async_remote_copy(
            src_ref=x_ref.at[pl.ds(0, 1), :],
            dst_ref=halo_ref.at[1],
            send_sem=send_sems.at[1],
            recv_sem=recv_sems.at[1],
            device_id=(my - 1,),
            device_id_type=pl.DeviceIdType.MESH,
        )

        @pl.when(jnp.logical_not(is_last))
        def _():
            send_right.start()

        @pl.when(jnp.logical_not(is_first))
        def _():
            send_left.start()

        out_ref[pl.ds(1, m - 2), :] = (
            0.25 * x_ref[pl.ds(0, m - 2), :]
            + 0.5 * x_ref[pl.ds(1, m - 2), :]
            + 0.25 * x_ref[pl.ds(2, m - 2), :]
        )

        @pl.when(is_first)
        def _():
            out_ref[pl.ds(0, 1), :] = x_ref[pl.ds(0, 1), :]

        @pl.when(jnp.logical_not(is_first))
        def _():
            send_right.wait_recv()
            out_ref[pl.ds(0, 1), :] = (
                0.25 * halo_ref[0]
                + 0.5 * x_ref[pl.ds(0, 1), :]
                + 0.25 * x_ref[pl.ds(1, 1), :]
            )

        @pl.when(is_last)
        def _():
            out_ref[pl.ds(m - 1, 1), :] = x_ref[pl.ds(m - 1, 1), :]

        @pl.when(jnp.logical_not(is_last))
        def _():
            send_left.wait_recv()
            out_ref[pl.ds(m - 1, 1), :] = (
                0.25 * x_ref[pl.ds(m - 2, 1), :]
                + 0.5 * x_ref[pl.ds(m - 1, 1), :]
                + 0.25 * halo_ref[1]
            )

        @pl.when(jnp.logical_not(is_last))
        def _():
            send_right.wait_send()

        @pl.when(jnp.logical_not(is_first))
        def _():
            send_left.wait_send()

    return pl.pallas_call(
        body,
        out_shape=jax.ShapeDtypeStruct((m, n), x.dtype),
        in_specs=[pl.BlockSpec(memory_space=pltpu.VMEM)],
        out_specs=pl.BlockSpec(memory_space=pltpu.VMEM),
        scratch_shapes=[
            pltpu.VMEM((2, 1, n), x.dtype),
            pltpu.SemaphoreType.DMA((2,)),
            pltpu.SemaphoreType.DMA((2,)),
        ],
        compiler_params=pltpu.CompilerParams(collective_id=0),
    )(x)
